# baseline (speedup 1.0000x reference)
import functools
import math

import jax
import jax.numpy as jnp
from jax import lax
from jax.experimental import pallas as pl
from jax.experimental.pallas import tpu as pltpu
from jax.experimental.pallas import tpu_sc as plsc

N = 4096
DIM = 128
KSEL = 2048
ALPHA = 0.1
LAMDA = 0.5
CH = 512

_INTERPRET = False


def _select_kernel(s_ref, h_ref, newh_ref, idx_ref, cum_ref):
    s = s_ref[...]
    u = jax.lax.bitcast_convert_type(s, jnp.int32)

    def body_t(i, t):
        b = 30 - i
        cand = t | (jnp.int32(1) << b)
        cnt = jnp.sum((u >= cand).astype(jnp.int32))
        return jax.lax.select(cnt >= KSEL, cand, t)

    t = jax.lax.fori_loop(0, 31, body_t, jnp.int32(0))
    cnt_gt = jnp.sum((u > t).astype(jnp.int32))
    need = KSEL - cnt_gt
    eq = u == t
    gidx = jax.lax.broadcasted_iota(jnp.int32, s.shape, 0)

    def body_c(i, c):
        b = 12 - i
        cand = c | (jnp.int32(1) << b)
        f = jnp.sum((eq & (gidx < cand)).astype(jnp.int32))
        return jax.lax.select(f <= need, cand, c)

    c = jax.lax.fori_loop(0, 13, body_c, jnp.int32(0))
    m = (u > t) | (eq & (gidx < c))
    mf = m.astype(jnp.float32)
    newh_ref[...] = h_ref[...] * (s * mf)

    for ic in range(N // CH):
        acc = jnp.zeros((1, CH), jnp.float32)
        i_row = jax.lax.broadcasted_iota(jnp.int32, (1, CH), 1) + ic * CH
        for jc in range(N // CH):
            j_col = jax.lax.broadcasted_iota(jnp.int32, (CH, 1), 0) + jc * CH
            mj = mf[jc * CH:(jc + 1) * CH, :]
            contrib = jnp.where(j_col <= i_row, mj, 0.0)
            acc = acc + jnp.sum(contrib, axis=0, keepdims=True)
        cum_ref[0, ic * CH:(ic + 1) * CH] = acc[0, :]

    cum = cum_ref[...]
    for qc in range(KSEL // CH):
        q_col = (jax.lax.broadcasted_iota(jnp.int32, (CH, 1), 0)
                 + qc * CH).astype(jnp.float32)
        le = (cum <= q_col).astype(jnp.float32)
        idx_ref[qc * CH:(qc + 1) * CH, :] = (
            jnp.sum(le, axis=1, keepdims=True).astype(jnp.int32))


def _bintrans_kernel(e_ref, at_ref):
    at_ref[...] = (e_ref[...].T != 0.0).astype(jnp.bfloat16)


def _make_sc_gather():
    info = plsc.get_sparse_core_info()
    nw = info.num_cores * info.num_subcores
    b_per_w = KSEL // nw
    chunk = 16
    nchunks = b_per_w // chunk
    mesh = plsc.VectorSubcoreMesh(core_axis_name="c", subcore_axis_name="s")

    @functools.partial(
        pl.kernel, mesh=mesh,
        out_type=(jax.ShapeDtypeStruct((KSEL, N), jnp.float32),
                  jax.ShapeDtypeStruct((KSEL, N // 2), jnp.float32)),
        scratch_types=[
            pltpu.VMEM((chunk,), jnp.int32),
            pltpu.VMEM((chunk, N), jnp.float32),
            pltpu.VMEM((chunk, N // 2), jnp.float32),
            pltpu.SemaphoreType.DMA,
            pltpu.SemaphoreType.DMA,
        ],
    )
    def sc_gather(edge_hbm, atf_hbm, idx_hbm, er_hbm, act_hbm,
                  idx_v, erows_v, atrows_v, sem_e, sem_a):
        wid = lax.axis_index("s") * info.num_cores + lax.axis_index("c")
        base = wid * b_per_w
        for c in range(nchunks):
            off = base + c * chunk
            pltpu.sync_copy(idx_hbm.at[pl.ds(off, chunk)], idx_v)
            cp_e = pltpu.async_copy(edge_hbm.at[idx_v], erows_v, sem_e)
            cp_a = pltpu.async_copy(atf_hbm.at[idx_v], atrows_v, sem_a)
            cp_e.wait()
            cp_a.wait()
            pltpu.sync_copy(erows_v, er_hbm.at[pl.ds(off, chunk)])
            pltpu.sync_copy(atrows_v, act_hbm.at[pl.ds(off, chunk)])

    return sc_gather


def _sq_kernel(l_ref, r_ref, g_ref):
    lb = (l_ref[...] != 0.0).astype(jnp.bfloat16)
    acc = jax.lax.dot_general(
        lb, r_ref[...], (((1,), (1,)), ((), ())),
        preferred_element_type=jnp.float32)
    g_ref[...] = (acc != 0.0).astype(jnp.bfloat16)


def _gcn_kernel(g_ref, h0_ref, w_ref, out_ref):
    g = g_ref[...]
    h0 = h0_ref[...]
    dn = (((1,), (0,)), ((), ()))
    h = h0
    for layer in range(1, 4):
        theta = math.log(LAMDA / layer + 1.0)
        ha = h.astype(jnp.bfloat16)
        hb = (h - ha.astype(jnp.float32)).astype(jnp.bfloat16)
        hi = (jax.lax.dot_general(g, ha, dn, preferred_element_type=jnp.float32)
              + jax.lax.dot_general(g, hb, dn, preferred_element_type=jnp.float32))
        w = w_ref[layer - 1]
        sup = (jax.lax.dot_general(hi, w[:DIM], dn,
                                   preferred_element_type=jnp.float32)
               + jax.lax.dot_general(h0, w[DIM:], dn,
                                     preferred_element_type=jnp.float32))
        r = (1.0 - ALPHA) * hi + ALPHA * h0
        out = theta * sup + (1.0 - theta) * r + h
        h = jnp.maximum(h + out, 0.0)
    out_ref[...] = h


def _scatter_kernel(idxf_ref, h_ref, out_ref, *, block):
    i = pl.program_id(0)
    p_col = (jax.lax.broadcasted_iota(jnp.int32, (block, 1), 0)
             + i * block).astype(jnp.float32)
    oh = (idxf_ref[...] == p_col).astype(jnp.float32)
    out_ref[...] = jax.lax.dot_general(
        oh, h_ref[...], (((1,), (0,)), ((), ())),
        preferred_element_type=jnp.float32)


def _hgather_kernel(idxc_ref, h_ref, out_ref, *, block):
    i_row = jax.lax.broadcasted_iota(jnp.int32, (1, N), 1).astype(jnp.float32)
    oh = (idxc_ref[...].astype(jnp.float32) == i_row).astype(jnp.float32)
    out_ref[...] = jax.lax.dot_general(
        oh, h_ref[...], (((1,), (0,)), ((), ())),
        preferred_element_type=jnp.float32)


def kernel(feat, edge, ep, fc_w, fc_b, proj_w, proj_b, gcn_w):
    f32 = jnp.float32
    bf16 = jnp.bfloat16

    h = jax.nn.relu(feat @ fc_w.T + fc_b)
    weights = (h @ proj_w.T + proj_b).squeeze()
    scores = jax.nn.sigmoid(weights).reshape(N, 1)

    new_h, idxc = pl.pallas_call(
        _select_kernel,
        out_shape=(jax.ShapeDtypeStruct((N, DIM), f32),
                   jax.ShapeDtypeStruct((KSEL, 1), jnp.int32)),
        scratch_shapes=[pltpu.VMEM((1, N), f32)],
        interpret=_INTERPRET,
    )(scores, h)
    idx1d = idxc.reshape(KSEL)
    idxf_row = idxc.reshape(1, KSEL).astype(f32)

    BT = 512
    at = pl.pallas_call(
        _bintrans_kernel,
        grid=(N // BT, N // BT),
        in_specs=[pl.BlockSpec((BT, BT), lambda i, j: (i, j))],
        out_specs=pl.BlockSpec((BT, BT), lambda i, j: (j, i)),
        out_shape=jax.ShapeDtypeStruct((N, N), bf16),
        interpret=_INTERPRET,
    )(edge)

    atf = jax.lax.bitcast_convert_type(at.reshape(N, N // 2, 2), f32)
    er, actf = _make_sc_gather()(edge, atf, idx1d)
    act = jax.lax.bitcast_convert_type(actf, bf16).reshape(KSEL, N)

    BM, BN = 1024, 512
    g = pl.pallas_call(
        _sq_kernel,
        grid=(KSEL // BM, KSEL // BN),
        in_specs=[
            pl.BlockSpec((BM, N), lambda i, j: (i, 0)),
            pl.BlockSpec((BN, N), lambda i, j: (j, 0)),
        ],
        out_specs=pl.BlockSpec((BM, BN), lambda i, j: (i, j)),
        out_shape=jax.ShapeDtypeStruct((KSEL, KSEL), bf16),
        interpret=_INTERPRET,
    )(er, act)

    BH = 512
    h0c = pl.pallas_call(
        lambda i_ref, h_ref, o_ref: _hgather_kernel(i_ref, h_ref, o_ref, block=BH),
        grid=(KSEL // BH,),
        in_specs=[
            pl.BlockSpec((BH, 1), lambda i: (i, 0)),
            pl.BlockSpec((N, DIM), lambda i: (0, 0)),
        ],
        out_specs=pl.BlockSpec((BH, DIM), lambda i: (i, 0)),
        out_shape=jax.ShapeDtypeStruct((KSEL, DIM), f32),
        interpret=_INTERPRET,
    )(idxc.astype(f32), new_h)

    h3 = pl.pallas_call(
        _gcn_kernel,
        out_shape=jax.ShapeDtypeStruct((KSEL, DIM), f32),
        interpret=_INTERPRET,
    )(g, h0c, gcn_w)

    BO = 512
    out = pl.pallas_call(
        lambda i_ref, h_ref, o_ref: _scatter_kernel(i_ref, h_ref, o_ref, block=BO),
        grid=(N // BO,),
        in_specs=[
            pl.BlockSpec((1, KSEL), lambda i: (0, 0)),
            pl.BlockSpec((KSEL, DIM), lambda i: (0, 0)),
        ],
        out_specs=pl.BlockSpec((BO, DIM), lambda i: (i, 0)),
        out_shape=jax.ShapeDtypeStruct((N, DIM), f32),
        interpret=_INTERPRET,
    )(idxf_row, h3)

    return out


# speedup vs baseline: 2.3829x; 2.3829x over previous
import math

import jax
import jax.numpy as jnp
from jax.experimental import pallas as pl
from jax.experimental.pallas import tpu as pltpu

N = 4096
DIM = 128
KSEL = 2048
ALPHA = 0.1
LAMDA = 0.5

_INTERPRET = False


def _select_kernel(s_ref, h_ref, newh_ref, mask_ref):
    s = s_ref[...]
    u = jax.lax.bitcast_convert_type(s, jnp.int32)

    def body_t(i, t):
        b = 30 - i
        cand = t | (jnp.int32(1) << b)
        cnt = jnp.sum((u >= cand).astype(jnp.int32))
        return jax.lax.select(cnt >= KSEL, cand, t)

    t = jax.lax.fori_loop(0, 31, body_t, jnp.int32(0))
    cnt_gt = jnp.sum((u > t).astype(jnp.int32))
    need = KSEL - cnt_gt
    eq = u == t
    gidx = jax.lax.broadcasted_iota(jnp.int32, s.shape, 0)

    def body_c(i, c):
        b = 12 - i
        cand = c | (jnp.int32(1) << b)
        f = jnp.sum((eq & (gidx < cand)).astype(jnp.int32))
        return jax.lax.select(f <= need, cand, c)

    c = jax.lax.fori_loop(0, 13, body_c, jnp.int32(0))
    m = (u > t) | (eq & (gidx < c))
    mf = m.astype(jnp.float32)
    mask_ref[...] = mf
    newh_ref[...] = h_ref[...] * (s * mf)


def _bin_kernel(e_ref, a_ref):
    a_ref[...] = (e_ref[...] != 0.0).astype(jnp.int8)


def _sq_kernel(l_ref, r_ref, mr_ref, mc_ref, g_ref):
    acc = jax.lax.dot_general(
        l_ref[...], r_ref[...], (((1,), (0,)), ((), ())),
        preferred_element_type=jnp.int32)
    nz = (acc != 0) & (mr_ref[...] != 0.0) & (mc_ref[...] != 0.0)
    g_ref[...] = nz.astype(jnp.bfloat16)


def _gcn_kernel(theta, g_ref, hfull_ref, hrow_ref, h0row_ref, w_ref, out_ref):
    hf = hfull_ref[...]
    ha = hf.astype(jnp.bfloat16)
    hb = (hf - ha.astype(jnp.float32)).astype(jnp.bfloat16)
    hcat = jnp.concatenate([ha, hb], axis=1)
    dn = (((1,), (0,)), ((), ()))
    hi2 = jax.lax.dot_general(g_ref[...], hcat, dn,
                              preferred_element_type=jnp.float32)
    hi = hi2[:, :DIM] + hi2[:, DIM:]
    h0r = h0row_ref[...]
    hr = hrow_ref[...]
    sup = jax.lax.dot_general(
        jnp.concatenate([hi, h0r], axis=1), w_ref[...], dn,
        preferred_element_type=jnp.float32)
    r = (1.0 - ALPHA) * hi + ALPHA * h0r
    out = theta * sup + (1.0 - theta) * r + hr
    out_ref[...] = jnp.maximum(hr + out, 0.0)


def kernel(feat, edge, ep, fc_w, fc_b, proj_w, proj_b, gcn_w):
    f32 = jnp.float32
    bf16 = jnp.bfloat16

    h = jax.nn.relu(feat @ fc_w.T + fc_b)
    weights = (h @ proj_w.T + proj_b).squeeze()
    scores = jax.nn.sigmoid(weights).reshape(N, 1)

    new_h, mask = pl.pallas_call(
        _select_kernel,
        out_shape=(jax.ShapeDtypeStruct((N, DIM), f32),
                   jax.ShapeDtypeStruct((N, 1), f32)),
        interpret=_INTERPRET,
    )(scores, h)
    mask_row = mask.reshape(1, N)

    BA = 512
    abin = pl.pallas_call(
        _bin_kernel,
        grid=(N // BA,),
        in_specs=[pl.BlockSpec((BA, N), lambda i: (i, 0))],
        out_specs=pl.BlockSpec((BA, N), lambda i: (i, 0)),
        out_shape=jax.ShapeDtypeStruct((N, N), jnp.int8),
        interpret=_INTERPRET,
    )(edge)

    BM, BN = 1024, 1024
    g = pl.pallas_call(
        _sq_kernel,
        grid=(N // BM, N // BN),
        in_specs=[
            pl.BlockSpec((BM, N), lambda i, j: (i, 0)),
            pl.BlockSpec((N, BN), lambda i, j: (0, j)),
            pl.BlockSpec((BM, 1), lambda i, j: (i, 0)),
            pl.BlockSpec((1, BN), lambda i, j: (0, j)),
        ],
        out_specs=pl.BlockSpec((BM, BN), lambda i, j: (i, j)),
        out_shape=jax.ShapeDtypeStruct((N, N), bf16),
        interpret=_INTERPRET,
    )(abin, abin, mask, mask_row)

    BG = 1024
    hcur = new_h
    for layer in range(1, 4):
        theta = math.log(LAMDA / layer + 1.0)
        hcur = pl.pallas_call(
            lambda g_ref, hf, hr, h0, w, o, _t=theta: _gcn_kernel(
                _t, g_ref, hf, hr, h0, w, o),
            grid=(N // BG,),
            in_specs=[
                pl.BlockSpec((BG, N), lambda i: (i, 0)),
                pl.BlockSpec((N, DIM), lambda i: (0, 0)),
                pl.BlockSpec((BG, DIM), lambda i: (i, 0)),
                pl.BlockSpec((BG, DIM), lambda i: (i, 0)),
                pl.BlockSpec((2 * DIM, DIM), lambda i: (0, 0)),
            ],
            out_specs=pl.BlockSpec((BG, DIM), lambda i: (i, 0)),
            out_shape=jax.ShapeDtypeStruct((N, DIM), f32),
            interpret=_INTERPRET,
        )(g, hcur, hcur, new_h, gcn_w[layer - 1])

    return hcur


# speedup vs baseline: 2.3947x; 1.0049x over previous
import math

import jax
import jax.numpy as jnp
from jax.experimental import pallas as pl
from jax.experimental.pallas import tpu as pltpu

N = 4096
DIM = 128
KSEL = 2048
ALPHA = 0.1
LAMDA = 0.5

_INTERPRET = False


def _select_kernel(s_ref, h_ref, newh_ref, mask_ref):
    s = s_ref[...]
    u = jax.lax.bitcast_convert_type(s, jnp.int32)

    def body_t(i, t):
        b = 30 - i
        cand = t | (jnp.int32(1) << b)
        cnt = jnp.sum((u >= cand).astype(jnp.int32))
        return jax.lax.select(cnt >= KSEL, cand, t)

    t = jax.lax.fori_loop(0, 31, body_t, jnp.int32(0))
    cnt_gt = jnp.sum((u > t).astype(jnp.int32))
    need = KSEL - cnt_gt
    eq = u == t
    gidx = jax.lax.broadcasted_iota(jnp.int32, s.shape, 0)

    def body_c(i, c):
        b = 12 - i
        cand = c | (jnp.int32(1) << b)
        f = jnp.sum((eq & (gidx < cand)).astype(jnp.int32))
        return jax.lax.select(f <= need, cand, c)

    c = jax.lax.fori_loop(0, 13, body_c, jnp.int32(0))
    m = (u > t) | (eq & (gidx < c))
    mf = m.astype(jnp.float32)
    mask_ref[...] = mf
    newh_ref[...] = h_ref[...] * (s * mf)


def _bin_kernel(e_ref, a_ref):
    a_ref[...] = (e_ref[...] != 0.0).astype(jnp.int8)


def _sq_kernel(l_ref, r_ref, mr_ref, mc_ref, g_ref):
    acc = jax.lax.dot_general(
        l_ref[...], r_ref[...], (((1,), (0,)), ((), ())),
        preferred_element_type=jnp.int32)
    nz = (acc != 0) & (mr_ref[...] != 0.0) & (mc_ref[...] != 0.0)
    g_ref[...] = nz.astype(jnp.int8)


def _gcn_kernel(theta, g_ref, hfull_ref, hrow_ref, h0row_ref, w_ref, out_ref):
    hf = hfull_ref[...]
    ha = hf.astype(jnp.bfloat16)
    hb = (hf - ha.astype(jnp.float32)).astype(jnp.bfloat16)
    hcat = jnp.concatenate([ha, hb], axis=1)
    dn = (((1,), (0,)), ((), ()))
    hi2 = jax.lax.dot_general(g_ref[...].astype(jnp.bfloat16), hcat, dn,
                              preferred_element_type=jnp.float32)
    hi = hi2[:, :DIM] + hi2[:, DIM:]
    h0r = h0row_ref[...]
    hr = hrow_ref[...]
    sup = jax.lax.dot_general(
        jnp.concatenate([hi, h0r], axis=1), w_ref[...], dn,
        preferred_element_type=jnp.float32)
    r = (1.0 - ALPHA) * hi + ALPHA * h0r
    out = theta * sup + (1.0 - theta) * r + hr
    out_ref[...] = jnp.maximum(hr + out, 0.0)


def kernel(feat, edge, ep, fc_w, fc_b, proj_w, proj_b, gcn_w):
    f32 = jnp.float32
    bf16 = jnp.bfloat16

    h = jax.nn.relu(feat @ fc_w.T + fc_b)
    weights = (h @ proj_w.T + proj_b).squeeze()
    scores = jax.nn.sigmoid(weights).reshape(N, 1)

    new_h, mask = pl.pallas_call(
        _select_kernel,
        out_shape=(jax.ShapeDtypeStruct((N, DIM), f32),
                   jax.ShapeDtypeStruct((N, 1), f32)),
        interpret=_INTERPRET,
    )(scores, h)
    mask_row = mask.reshape(1, N)

    BA = 512
    abin = pl.pallas_call(
        _bin_kernel,
        grid=(N // BA,),
        in_specs=[pl.BlockSpec((BA, N), lambda i: (i, 0))],
        out_specs=pl.BlockSpec((BA, N), lambda i: (i, 0)),
        out_shape=jax.ShapeDtypeStruct((N, N), jnp.int8),
        interpret=_INTERPRET,
    )(edge)

    BM, BN = 2048, 1024
    g = pl.pallas_call(
        _sq_kernel,
        grid=(N // BM, N // BN),
        in_specs=[
            pl.BlockSpec((BM, N), lambda i, j: (i, 0)),
            pl.BlockSpec((N, BN), lambda i, j: (0, j)),
            pl.BlockSpec((BM, 1), lambda i, j: (i, 0)),
            pl.BlockSpec((1, BN), lambda i, j: (0, j)),
        ],
        out_specs=pl.BlockSpec((BM, BN), lambda i, j: (i, j)),
        out_shape=jax.ShapeDtypeStruct((N, N), jnp.int8),
        interpret=_INTERPRET,
    )(abin, abin, mask, mask_row)

    BG = 1024
    hcur = new_h
    for layer in range(1, 4):
        theta = math.log(LAMDA / layer + 1.0)
        hcur = pl.pallas_call(
            lambda g_ref, hf, hr, h0, w, o, _t=theta: _gcn_kernel(
                _t, g_ref, hf, hr, h0, w, o),
            grid=(N // BG,),
            in_specs=[
                pl.BlockSpec((BG, N), lambda i: (i, 0)),
                pl.BlockSpec((N, DIM), lambda i: (0, 0)),
                pl.BlockSpec((BG, DIM), lambda i: (i, 0)),
                pl.BlockSpec((BG, DIM), lambda i: (i, 0)),
                pl.BlockSpec((2 * DIM, DIM), lambda i: (0, 0)),
            ],
            out_specs=pl.BlockSpec((BG, DIM), lambda i: (i, 0)),
            out_shape=jax.ShapeDtypeStruct((N, DIM), f32),
            interpret=_INTERPRET,
        )(g, hcur, hcur, new_h, gcn_w[layer - 1])

    return hcur


# speedup vs baseline: 3.0416x; 1.2702x over previous
import math

import jax
import jax.numpy as jnp
from jax.experimental import pallas as pl
from jax.experimental.pallas import tpu as pltpu

N = 4096
DIM = 128
KSEL = 2048
ALPHA = 0.1
LAMDA = 0.5

_INTERPRET = False


def _select_kernel(s_ref, h_ref, newh_ref, mask_ref):
    s = s_ref[...]
    u = jax.lax.bitcast_convert_type(s, jnp.int32)

    def body_t(i, t):
        b = 30 - i
        cand = t | (jnp.int32(1) << b)
        cnt = jnp.sum((u >= cand).astype(jnp.int32))
        return jax.lax.select(cnt >= KSEL, cand, t)

    t = jax.lax.fori_loop(0, 31, body_t, jnp.int32(0))
    cnt_gt = jnp.sum((u > t).astype(jnp.int32))
    need = KSEL - cnt_gt
    eq = u == t
    gidx = jax.lax.broadcasted_iota(jnp.int32, s.shape, 0)

    def body_c(i, c):
        b = 12 - i
        cand = c | (jnp.int32(1) << b)
        f = jnp.sum((eq & (gidx < cand)).astype(jnp.int32))
        return jax.lax.select(f <= need, cand, c)

    c = jax.lax.fori_loop(0, 13, body_c, jnp.int32(0))
    m = (u > t) | (eq & (gidx < c))
    mf = m.astype(jnp.float32)
    mask_ref[...] = mf
    newh_ref[...] = h_ref[...] * (s * mf)


def _bin_kernel(e_ref, a_ref):
    a_ref[...] = (e_ref[...] != 0.0).astype(jnp.float8_e4m3fn)


def _sq_kernel(l_ref, r_ref, mr_ref, mc_ref, g_ref):
    acc = jax.lax.dot_general(
        l_ref[...], r_ref[...], (((1,), (0,)), ((), ())),
        preferred_element_type=jnp.float32)
    nz = (acc != 0.0) & (mr_ref[...] != 0.0) & (mc_ref[...] != 0.0)
    g_ref[...] = nz.astype(jnp.int8)


def _gcn_kernel(theta, g_ref, hfull_ref, hrow_ref, h0row_ref, w_ref, out_ref):
    hf = hfull_ref[...]
    ha = hf.astype(jnp.bfloat16)
    hb = (hf - ha.astype(jnp.float32)).astype(jnp.bfloat16)
    hcat = jnp.concatenate([ha, hb], axis=1)
    dn = (((1,), (0,)), ((), ()))
    hi2 = jax.lax.dot_general(g_ref[...].astype(jnp.bfloat16), hcat, dn,
                              preferred_element_type=jnp.float32)
    hi = hi2[:, :DIM] + hi2[:, DIM:]
    h0r = h0row_ref[...]
    hr = hrow_ref[...]
    sup = jax.lax.dot_general(
        jnp.concatenate([hi, h0r], axis=1), w_ref[...], dn,
        preferred_element_type=jnp.float32)
    r = (1.0 - ALPHA) * hi + ALPHA * h0r
    out = theta * sup + (1.0 - theta) * r + hr
    out_ref[...] = jnp.maximum(hr + out, 0.0)


def kernel(feat, edge, ep, fc_w, fc_b, proj_w, proj_b, gcn_w):
    f32 = jnp.float32
    bf16 = jnp.bfloat16

    h = jax.nn.relu(feat @ fc_w.T + fc_b)
    weights = (h @ proj_w.T + proj_b).squeeze()
    scores = jax.nn.sigmoid(weights).reshape(N, 1)

    new_h, mask = pl.pallas_call(
        _select_kernel,
        out_shape=(jax.ShapeDtypeStruct((N, DIM), f32),
                   jax.ShapeDtypeStruct((N, 1), f32)),
        interpret=_INTERPRET,
    )(scores, h)
    mask_row = mask.reshape(1, N)

    BA = 512
    abin = pl.pallas_call(
        _bin_kernel,
        grid=(N // BA,),
        in_specs=[pl.BlockSpec((BA, N), lambda i: (i, 0))],
        out_specs=pl.BlockSpec((BA, N), lambda i: (i, 0)),
        out_shape=jax.ShapeDtypeStruct((N, N), jnp.float8_e4m3fn),
        interpret=_INTERPRET,
    )(edge)

    BM, BN = 2048, 1024
    g = pl.pallas_call(
        _sq_kernel,
        grid=(N // BM, N // BN),
        in_specs=[
            pl.BlockSpec((BM, N), lambda i, j: (i, 0)),
            pl.BlockSpec((N, BN), lambda i, j: (0, j)),
            pl.BlockSpec((BM, 1), lambda i, j: (i, 0)),
            pl.BlockSpec((1, BN), lambda i, j: (0, j)),
        ],
        out_specs=pl.BlockSpec((BM, BN), lambda i, j: (i, j)),
        out_shape=jax.ShapeDtypeStruct((N, N), jnp.int8),
        interpret=_INTERPRET,
    )(abin, abin, mask, mask_row)

    BG = 1024
    hcur = new_h
    for layer in range(1, 4):
        theta = math.log(LAMDA / layer + 1.0)
        hcur = pl.pallas_call(
            lambda g_ref, hf, hr, h0, w, o, _t=theta: _gcn_kernel(
                _t, g_ref, hf, hr, h0, w, o),
            grid=(N // BG,),
            in_specs=[
                pl.BlockSpec((BG, N), lambda i: (i, 0)),
                pl.BlockSpec((N, DIM), lambda i: (0, 0)),
                pl.BlockSpec((BG, DIM), lambda i: (i, 0)),
                pl.BlockSpec((BG, DIM), lambda i: (i, 0)),
                pl.BlockSpec((2 * DIM, DIM), lambda i: (0, 0)),
            ],
            out_specs=pl.BlockSpec((BG, DIM), lambda i: (i, 0)),
            out_shape=jax.ShapeDtypeStruct((N, DIM), f32),
            interpret=_INTERPRET,
        )(g, hcur, hcur, new_h, gcn_w[layer - 1])

    return hcur
